# Initial kernel scaffold; baseline (speedup 1.0000x reference)
import functools

import jax
import jax.numpy as jnp
from jax import lax
from jax.experimental import pallas as pl
from jax.experimental.pallas import tpu as pltpu
from jax.experimental.pallas import tpu_sc as plsc

N = 50000
DIM = 100
E = 800000
M = 2 * E
DP = 128
CC = 12544
NCHUNK = 4
NP = NCHUNK * CC
BLK = 128
NSEG = NCHUNK * 4
MP = M + NSEG * BLK
NBLK = MP // BLK
DUMP = CC
ACC_ROWS = CC + 256
ZROWS = ACC_ROWS // 16
WROWS = CC // 16
MMB = 512


def _mm_body(h_ref, w_ref, b_ref, o_ref, *, relu):
    h = h_ref[...]
    if relu:
        h = jnp.maximum(h, 0.0)
    acc = lax.dot_general(h, w_ref[...], (((1,), (1,)), ((), ())),
                          preferred_element_type=jnp.float32)
    o_ref[...] = acc + b_ref[...]


def _linear(h, w_pad, b_pad, relu):
    return pl.pallas_call(
        functools.partial(_mm_body, relu=relu),
        grid=(NP // MMB,),
        in_specs=[pl.BlockSpec((MMB, DP), lambda i: (i, 0)),
                  pl.BlockSpec((DP, DP), lambda i: (0, 0)),
                  pl.BlockSpec((1, DP), lambda i: (0, 0))],
        out_specs=pl.BlockSpec((MMB, DP), lambda i: (i, 0)),
        out_shape=jax.ShapeDtypeStruct((NP, DP), jnp.float32),
    )(h, w_pad, b_pad)


def _lane(vref, t):
    v = vref[pl.ds((t // 16) * 16, 16)]
    ii = lax.iota(jnp.int32, 16)
    return jnp.max(jnp.where(ii == (t % 16), v, 0))


def _agg_body(t_hbm, idx_hbm, jbs_hbm, out_hbm,
              jbs_v, idx_v, rows_v, zbuf, acc, sem, *, layer):
    core = lax.axis_index("c")
    sid = lax.axis_index("s")
    pltpu.sync_copy(jbs_hbm, jbs_v)

    def _zrow(r, carry):
        for k in range(DP // 16):
            zbuf[r, pl.ds(k * 16, 16)] = jnp.zeros((16,), jnp.float32)
        return carry
    lax.fori_loop(0, 128, _zrow, 0)

    for c in range(NCHUNK):
        @pl.when(core == (c % 2))
        def _chunk(c=c):
            zoff = sid * ZROWS
            off = 0
            for sz in [128] * 6 + [32]:
                pltpu.sync_copy(zbuf.at[pl.ds(0, sz)],
                                acc.at[pl.ds(zoff + off, sz)])
                off += sz
            plsc.subcore_barrier()

            jb1 = _lane(jbs_v, 4 * c + (3 - layer))

            def _blk(jb):
                pltpu.sync_copy(idx_hbm.at[jb], idx_v)
                pltpu.async_copy(t_hbm.at[idx_v.at[0]], rows_v, sem).wait()
                pltpu.sync_copy(rows_v, acc.at[idx_v.at[1]], add=True)
                return jb + 16

            lax.while_loop(lambda jb: jb < jb1, _blk,
                           _lane(jbs_v, 4 * c) + sid)
            plsc.subcore_barrier()

            woff = sid * WROWS
            off = 0
            for sz in [128] * 6 + [16]:
                pltpu.sync_copy(acc.at[pl.ds(woff + off, sz)],
                                out_hbm.at[pl.ds(c * CC + woff + off, sz)])
                off += sz
            plsc.subcore_barrier()


def _aggregate(t, idx_inter, jbs, layer):
    mesh = plsc.VectorSubcoreMesh(core_axis_name="c", subcore_axis_name="s")
    kfn = pl.kernel(
        functools.partial(_agg_body, layer=layer),
        out_type=jax.ShapeDtypeStruct((NP, DP), jnp.float32),
        mesh=mesh,
        scratch_types=[
            pltpu.VMEM((32,), jnp.int32),
            pltpu.VMEM((2, BLK), jnp.int32),
            pltpu.VMEM((BLK, DP), jnp.float32),
            pltpu.VMEM((128, DP), jnp.float32),
            pltpu.VMEM_SHARED((ACC_ROWS, DP), jnp.float32),
            pltpu.SemaphoreType.DMA,
        ],
    )
    return kfn(t, idx_inter, jbs)


def kernel(x, edge_index, node_depth, W0, b0, W1, b1, W2, b2):
    ei = edge_index.astype(jnp.int32)
    depth = node_depth.astype(jnp.int32)
    u, v = ei[0], ei[1]
    recv = jnp.concatenate([u, v])
    send = jnp.concatenate([v, u])
    dr = jnp.minimum(depth[recv], 3)
    key = (recv // CC) * 4 + dr
    sk, order = lax.sort_key_val(key, jnp.arange(M, dtype=jnp.int32))
    r_s = recv[order]
    s_s = send[order]
    rloc = r_s - (r_s // CC) * CC
    starts = jnp.searchsorted(
        sk, jnp.arange(NSEG + 1, dtype=jnp.int32)).astype(jnp.int32)
    counts = starts[1:] - starts[:-1]
    cpad = ((counts + BLK - 1) // BLK) * BLK
    pstarts = jnp.concatenate(
        [jnp.zeros((1,), jnp.int32), jnp.cumsum(cpad).astype(jnp.int32)])
    j = jnp.arange(MP, dtype=jnp.int32)
    seg = jnp.searchsorted(pstarts, j, side="right").astype(jnp.int32) - 1
    seg = jnp.minimum(seg, NSEG - 1)
    padofs = pstarts[:-1] - starts[:-1]
    src = j - padofs[seg]
    valid = (src < starts[seg + 1]) & (j < pstarts[NSEG])
    srcc = jnp.clip(src, 0, M - 1)
    s_pad = jnp.where(valid, s_s[srcc], 0)
    r_pad = jnp.where(valid, rloc[srcc], DUMP)
    idx_inter = jnp.stack(
        [s_pad.reshape(NBLK, BLK), r_pad.reshape(NBLK, BLK)], axis=1)
    jbs = jnp.zeros((32,), jnp.int32).at[:NSEG + 1].set(
        (pstarts // BLK).astype(jnp.int32))

    xp = jnp.pad(x, ((0, NP - N), (0, DP - DIM)))
    params = [(W0, b0), (W1, b1), (W2, b2)]
    h = xp
    for l in range(3):
        W, b = params[l]
        wp = jnp.pad(W, ((0, DP - DIM), (0, DP - DIM)))
        bp = jnp.pad(b, (0, DP - DIM)).reshape(1, DP)
        t = _linear(h, wp, bp, relu=(l > 0))
        h = _aggregate(t, idx_inter, jbs, l)
    return h[:N, :DIM]


# speedup vs baseline: 3.1802x; 3.1802x over previous
import functools

import jax
import jax.numpy as jnp
from jax import lax
from jax.experimental import pallas as pl
from jax.experimental.pallas import tpu as pltpu
from jax.experimental.pallas import tpu_sc as plsc

N = 50000
DIM = 100
E = 800000
M = 2 * E
DP = 128
CC = 12544
NCHUNK = 4
NP = NCHUNK * CC
BLK = 128
NSEG = NCHUNK * 4
MP = M + NSEG * BLK
NBLK = MP // BLK
DUMP = CC
ACC_ROWS = CC + 16
ZROWS = ACC_ROWS // 16
WROWS = CC // 16
MMB = 512


def _mm_body(h_ref, w_ref, b_ref, o_ref, *, relu):
    h = h_ref[...]
    if relu:
        h = jnp.maximum(h, 0.0)
    acc = lax.dot_general(h, w_ref[...], (((1,), (1,)), ((), ())),
                          preferred_element_type=jnp.float32)
    o_ref[...] = acc + b_ref[...]


def _linear(h, w_pad, b_pad, relu):
    return pl.pallas_call(
        functools.partial(_mm_body, relu=relu),
        grid=(NP // MMB,),
        in_specs=[pl.BlockSpec((MMB, DP), lambda i: (i, 0)),
                  pl.BlockSpec((DP, DP), lambda i: (0, 0)),
                  pl.BlockSpec((1, DP), lambda i: (0, 0))],
        out_specs=pl.BlockSpec((MMB, DP), lambda i: (i, 0)),
        out_shape=jax.ShapeDtypeStruct((NP, DP), jnp.float32),
    )(h, w_pad, b_pad)


def _lane(vref, t):
    return vref[pl.ds(0, 16)][t]


def _agg_body(t_hbm, idx_hbm, jbs_hbm, out_hbm,
              jbs_v, idx_v, rows_v, acc, sem, sem_i, sem_a, *, layer):
    core = lax.axis_index("c")
    sid = lax.axis_index("s")
    pltpu.async_copy(jbs_hbm, jbs_v, sem_i).wait()

    for c in range(NCHUNK):
        @pl.when(core == (c % 2))
        def _chunk(c=c):
            def _zrow(r, carry):
                for k in range(DP // 16):
                    rows_v[r, pl.ds(k * 16, 16)] = jnp.zeros(
                        (16,), jnp.float32)
                return carry
            lax.fori_loop(0, 128, _zrow, 0)
            zoff = sid * ZROWS
            off = 0
            for sz in [128] * 6 + [17]:
                pltpu.async_copy(rows_v.at[pl.ds(0, sz)],
                                 acc.at[pl.ds(zoff + off, sz)], sem_i).wait()
                off += sz
            plsc.subcore_barrier()

            jb0 = _lane(jbs_v, 4 * c) + sid
            jb1 = _lane(jbs_v, 4 * c + (3 - layer))
            nblk = jnp.maximum(jb1 - jb0 + 15, 0) // 16

            def _blk(i, carry):
                jb = jb0 + i * 16
                pltpu.async_copy(idx_hbm.at[jb], idx_v, sem_i).wait()
                pltpu.async_copy(t_hbm.at[idx_v.at[0]], rows_v, sem).wait()
                pltpu.async_copy(rows_v, acc.at[idx_v.at[1]], sem_a,
                                 add=True).wait()
                return carry

            lax.fori_loop(0, nblk, _blk, 0)
            plsc.subcore_barrier()

            woff = sid * WROWS
            off = 0
            for sz in [128] * 6 + [16]:
                pltpu.async_copy(acc.at[pl.ds(woff + off, sz)],
                                 out_hbm.at[pl.ds(c * CC + woff + off, sz)],
                                 sem_i).wait()
                off += sz
            plsc.subcore_barrier()


def _aggregate(t, idx_inter, jbs, layer):
    mesh = plsc.VectorSubcoreMesh(core_axis_name="c", subcore_axis_name="s")
    kfn = pl.kernel(
        functools.partial(_agg_body, layer=layer),
        out_type=jax.ShapeDtypeStruct((NP, DP), jnp.float32),
        mesh=mesh,
        scratch_types=[
            pltpu.VMEM((32,), jnp.int32),
            pltpu.VMEM((2, BLK), jnp.int32),
            pltpu.VMEM((BLK, DP), jnp.float32),
            pltpu.VMEM_SHARED((ACC_ROWS, DP), jnp.float32),
            pltpu.SemaphoreType.DMA,
            pltpu.SemaphoreType.DMA,
            pltpu.SemaphoreType.DMA,
        ],
    )
    return kfn(t, idx_inter, jbs)


def kernel(x, edge_index, node_depth, W0, b0, W1, b1, W2, b2):
    ei = edge_index.astype(jnp.int32)
    depth = node_depth.astype(jnp.int32)
    u, v = ei[0], ei[1]
    recv = jnp.concatenate([u, v])
    send = jnp.concatenate([v, u])
    dr = jnp.minimum(depth[recv], 3)
    key = (recv // CC) * 4 + dr
    sk, order = lax.sort_key_val(key, jnp.arange(M, dtype=jnp.int32))
    r_s = recv[order]
    s_s = send[order]
    rloc = r_s - (r_s // CC) * CC
    starts = jnp.searchsorted(
        sk, jnp.arange(NSEG + 1, dtype=jnp.int32)).astype(jnp.int32)
    counts = starts[1:] - starts[:-1]
    cpad = ((counts + BLK - 1) // BLK) * BLK
    pstarts = jnp.concatenate(
        [jnp.zeros((1,), jnp.int32), jnp.cumsum(cpad).astype(jnp.int32)])
    j = jnp.arange(MP, dtype=jnp.int32)
    seg = jnp.searchsorted(pstarts, j, side="right").astype(jnp.int32) - 1
    seg = jnp.minimum(seg, NSEG - 1)
    padofs = pstarts[:-1] - starts[:-1]
    src = j - padofs[seg]
    valid = (src < starts[seg + 1]) & (j < pstarts[NSEG])
    srcc = jnp.clip(src, 0, M - 1)
    s_pad = jnp.where(valid, s_s[srcc], 0)
    r_pad = jnp.where(valid, rloc[srcc], DUMP)
    idx_inter = jnp.stack(
        [s_pad.reshape(NBLK, BLK), r_pad.reshape(NBLK, BLK)], axis=1)
    jbs = jnp.zeros((32,), jnp.int32).at[:NSEG + 1].set(
        (pstarts // BLK).astype(jnp.int32))

    xp = jnp.pad(x, ((0, NP - N), (0, DP - DIM)))
    params = [(W0, b0), (W1, b1), (W2, b2)]
    h = xp
    for l in range(3):
        W, b = params[l]
        wp = jnp.pad(W, ((0, DP - DIM), (0, DP - DIM)))
        bp = jnp.pad(b, (0, DP - DIM)).reshape(1, DP)
        t = _linear(h, wp, bp, relu=(l > 0))
        h = _aggregate(t, idx_inter, jbs, l)
    return h[:N, :DIM]


# speedup vs baseline: 3.7093x; 1.1664x over previous
import functools

import jax
import jax.numpy as jnp
from jax import lax
from jax.experimental import pallas as pl
from jax.experimental.pallas import tpu as pltpu
from jax.experimental.pallas import tpu_sc as plsc

N = 50000
DIM = 100
E = 800000
M = 2 * E
DP = 128
CSH = 13
CC = 8192
NCHUNK = 7
NPL = 50176
NPO = NCHUNK * CC
BLK = 128
MCAP = M + NCHUNK * BLK
NBLK = MCAP // BLK
DUMP = CC
ACC_ROWS = CC + 16
ZROWS = ACC_ROWS // 16
WROWS = CC // 16
MMB = 512
NB = 3
T = 3125
B = 512


def _mm_body(h_ref, m_ref, w_ref, b_ref, o_ref, *, relu, mask):
    h = h_ref[...]
    if mask:
        h = h * m_ref[...]
    if relu:
        h = jnp.maximum(h, 0.0)
    acc = lax.dot_general(h, w_ref[...], (((1,), (1,)), ((), ())),
                          preferred_element_type=jnp.float32)
    o_ref[...] = acc + b_ref[...]


def _linear(h, mcol, w_pad, b_pad, relu, mask):
    return pl.pallas_call(
        functools.partial(_mm_body, relu=relu, mask=mask),
        grid=(NPL // MMB,),
        in_specs=[pl.BlockSpec((MMB, DP), lambda i: (i, 0)),
                  pl.BlockSpec((MMB, 1), lambda i: (i, 0)),
                  pl.BlockSpec((DP, DP), lambda i: (0, 0)),
                  pl.BlockSpec((1, DP), lambda i: (0, 0))],
        out_specs=pl.BlockSpec((MMB, DP), lambda i: (i, 0)),
        out_shape=jax.ShapeDtypeStruct((NPL, DP), jnp.float32),
    )(h, mcol, w_pad, b_pad)


def _mask_body(h_ref, m_ref, o_ref):
    o_ref[...] = h_ref[...] * m_ref[...]


def _mask_mul(h, mcol):
    return pl.pallas_call(
        _mask_body,
        grid=(NPL // MMB,),
        in_specs=[pl.BlockSpec((MMB, DP), lambda i: (i, 0)),
                  pl.BlockSpec((MMB, 1), lambda i: (i, 0))],
        out_specs=pl.BlockSpec((MMB, DP), lambda i: (i, 0)),
        out_shape=jax.ShapeDtypeStruct((NPL, DP), jnp.float32),
    )(h, mcol)


def _lane(vref, t):
    return vref[pl.ds(0, 16)][t]


def _agg_body(t_hbm, idx_hbm, jbs_hbm, out_hbm,
              jbs_v, idx_v, rows_v, acc,
              sem0, gs0, gs1, gs2, ss0, ss1, ss2, is0, is1, is2):
    core = lax.axis_index("c")
    sid = lax.axis_index("s")
    gsem = [gs0, gs1, gs2]
    ssem = [ss0, ss1, ss2]
    isem = [is0, is1, is2]
    pltpu.async_copy(jbs_hbm, jbs_v, sem0).wait()

    for c in range(NCHUNK):
        @pl.when(core == (c % 2))
        def _chunk(c=c):
            def _zrow(r, carry):
                for k in range(DP // 16):
                    rows_v[0, r, pl.ds(k * 16, 16)] = jnp.zeros(
                        (16,), jnp.float32)
                return carry
            lax.fori_loop(0, 128, _zrow, 0)
            zoff = sid * ZROWS
            off = 0
            for sz in [128] * 4 + [1]:
                pltpu.async_copy(rows_v.at[0, pl.ds(0, sz)],
                                 acc.at[pl.ds(zoff + off, sz)], sem0).wait()
                off += sz
            plsc.subcore_barrier()

            jb0 = _lane(jbs_v, c) + sid
            jb1 = _lane(jbs_v, c + 1)
            n = jnp.maximum(jb1 - jb0 + 15, 0) // 16

            for j in range(NB):
                @pl.when(j < n)
                def _pro(j=j):
                    pltpu.async_copy(idx_hbm.at[jb0 + j * 16],
                                     idx_v.at[j], isem[j]).wait()
                    pltpu.async_copy(
                        t_hbm.at[idx_v.at[j, 0]], rows_v.at[j], gsem[j])

            nround = ((n + NB - 1) * 21846) >> 16

            def _round(r, carry):
                for b in range(NB):
                    i = r * NB + b

                    @pl.when(i < n)
                    def _unit(b=b, i=i):
                        pltpu.make_async_copy(
                            t_hbm.at[idx_v.at[b, 0]], rows_v.at[b],
                            gsem[b]).wait()
                        pltpu.async_copy(
                            rows_v.at[b], acc.at[idx_v.at[b, 1]], ssem[b],
                            add=True).wait()

                        @pl.when(i + NB < n)
                        def _refill():
                            jb = jb0 + (i + NB) * 16
                            pltpu.async_copy(idx_hbm.at[jb], idx_v.at[b],
                                             isem[b]).wait()
                            pltpu.async_copy(
                                t_hbm.at[idx_v.at[b, 0]], rows_v.at[b],
                                gsem[b])
                return carry

            lax.fori_loop(0, nround, _round, 0)
            plsc.subcore_barrier()

            woff = sid * WROWS
            off = 0
            for sz in [128] * 4:
                pltpu.async_copy(acc.at[pl.ds(woff + off, sz)],
                                 out_hbm.at[pl.ds(c * CC + woff + off, sz)],
                                 sem0).wait()
                off += sz
            plsc.subcore_barrier()


def _aggregate(t, idx_inter, jbs):
    mesh = plsc.VectorSubcoreMesh(core_axis_name="c", subcore_axis_name="s")
    kfn = pl.kernel(
        _agg_body,
        out_type=jax.ShapeDtypeStruct((NPO, DP), jnp.float32),
        mesh=mesh,
        scratch_types=[
            pltpu.VMEM((16,), jnp.int32),
            pltpu.VMEM((NB, 2, BLK), jnp.int32),
            pltpu.VMEM((NB, BLK, DP), jnp.float32),
            pltpu.VMEM_SHARED((ACC_ROWS, DP), jnp.float32),
        ] + [pltpu.SemaphoreType.DMA] * 10,
    )
    return kfn(t, idx_inter, jbs)


def _prep(edge_index):
    ei = edge_index.astype(jnp.int32)
    u, v = ei[0], ei[1]
    recv = jnp.concatenate([u, v])
    send = jnp.concatenate([v, u])
    ck = recv >> CSH
    rl = recv & (CC - 1)
    ck2 = ck.reshape(T, B)
    oh = (ck2[None, :, :] == jnp.arange(8, dtype=jnp.int32)[:, None, None])
    ohi = oh.astype(jnp.int32)
    cum = jnp.cumsum(ohi, axis=2)
    tile_tot = cum[:, :, -1]
    tile_base = jnp.cumsum(tile_tot, axis=1) - tile_tot
    btot = tile_tot.sum(axis=1)
    bpad = ((btot + BLK - 1) // BLK) * BLK
    pstart = jnp.concatenate(
        [jnp.zeros((1,), jnp.int32), jnp.cumsum(bpad).astype(jnp.int32)])
    tbase = pstart[:8, None] + tile_base
    dest = (jnp.where(oh, cum - 1 + tbase[:, :, None], 0)
            .sum(axis=0).reshape(M))
    fs = (dest >> 7) * 256 + (dest & (BLK - 1))
    fill = jnp.tile(
        jnp.concatenate([jnp.zeros((BLK,), jnp.int32),
                         jnp.full((BLK,), DUMP, jnp.int32)]), NBLK)
    flat = fill.at[fs].set(send, unique_indices=True,
                           mode="promise_in_bounds")
    flat = flat.at[fs + BLK].set(rl, unique_indices=True,
                                 mode="promise_in_bounds")
    idx_inter = flat.reshape(NBLK, 2, BLK)
    jbs = jnp.zeros((16,), jnp.int32).at[:NCHUNK + 1].set(
        (pstart[:NCHUNK + 1] // BLK).astype(jnp.int32))
    return idx_inter, jbs


def kernel(x, edge_index, node_depth, W0, b0, W1, b1, W2, b2):
    idx_inter, jbs = _prep(edge_index)
    depth = node_depth.astype(jnp.int32)
    dpad = jnp.pad(depth, (0, NPL - N), constant_values=99)
    ones = jnp.ones((NPL, 1), jnp.float32)
    masks = [ones] + [
        (dpad <= 2 - l).astype(jnp.float32).reshape(NPL, 1)
        for l in range(3)]

    xp = jnp.pad(x, ((0, NPL - N), (0, DP - DIM)))
    params = [(W0, b0), (W1, b1), (W2, b2)]
    h = xp
    for l in range(3):
        W, b = params[l]
        wp = jnp.pad(W, ((0, DP - DIM), (0, DP - DIM)))
        bp = jnp.pad(b, (0, DP - DIM)).reshape(1, DP)
        t = _linear(h, masks[l], wp, bp, relu=(l > 0), mask=(l > 0))
        h = _aggregate(t, idx_inter, jbs)
    out = _mask_mul(h, masks[3])
    return out[:N, :DIM]
